# baseline (speedup 1.0000x reference)
import jax
import jax.numpy as jnp
from jax.experimental import pallas as pl
from jax.experimental.pallas import tpu as pltpu

_NBUF = 4
_BMAX = 512


def _schedule(m):
    head = [64, 64, 128, 256]
    tail = [256, 128, 64, 64]
    mid = m - sum(head) - sum(tail)
    assert mid >= 0 and mid % _BMAX == 0
    return head + [_BMAX] * (mid // _BMAX) + tail


def _gate_kernel(x_hbm, w_hbm, o_hbm, buf, wbuf, obuf, sem_in, sem_out, sem_w):
    m, k = x_hbm.shape
    ch = _schedule(m)
    off = [0]
    for c in ch:
        off.append(off[-1] + c)

    def incopy(j, slot):
        return pltpu.make_async_copy(
            x_hbm.at[pl.ds(off[j], ch[j]), :],
            buf.at[slot, pl.ds(0, ch[j]), :],
            sem_in.at[slot])

    wcopy = pltpu.make_async_copy(w_hbm, wbuf, sem_w)
    wcopy.start()
    for s in range(min(_NBUF, len(ch))):
        incopy(s, s).start()
    wcopy.wait()

    outcopies = {}
    for j in range(len(ch)):
        slot = j % _NBUF
        oslot = j % 2
        incopy(j, slot).wait()
        if j >= 2:
            outcopies.pop(j - 2).wait()
        obuf[oslot, pl.ds(0, ch[j]), :] = jax.lax.dot_general(
            buf[slot, pl.ds(0, ch[j]), :], wbuf[...],
            (((1,), (1,)), ((), ())),
            precision=jax.lax.Precision.DEFAULT,
            preferred_element_type=jnp.float32)
        oc = pltpu.make_async_copy(
            obuf.at[oslot, pl.ds(0, ch[j]), :],
            o_hbm.at[pl.ds(off[j], ch[j]), :],
            sem_out.at[oslot])
        oc.start()
        outcopies[j] = oc
        nxt = j + _NBUF
        if nxt < len(ch):
            incopy(nxt, slot).start()
    for oc in outcopies.values():
        oc.wait()


def kernel(hidden_states, weight):
    m, k = hidden_states.shape
    e = weight.shape[0]
    return pl.pallas_call(
        _gate_kernel,
        in_specs=[
            pl.BlockSpec(memory_space=pltpu.HBM),
            pl.BlockSpec(memory_space=pltpu.HBM),
        ],
        out_specs=pl.BlockSpec(memory_space=pltpu.HBM),
        out_shape=jax.ShapeDtypeStruct((m, e), jnp.float32),
        scratch_shapes=[
            pltpu.VMEM((_NBUF, _BMAX, k), jnp.float32),
            pltpu.VMEM((e, k), jnp.float32),
            pltpu.VMEM((2, _BMAX, e), jnp.float32),
            pltpu.SemaphoreType.DMA((_NBUF,)),
            pltpu.SemaphoreType.DMA((2,)),
            pltpu.SemaphoreType.DMA,
        ],
        compiler_params=pltpu.CompilerParams(
            disable_bounds_checks=True,
            skip_device_barrier=True),
    )(hidden_states, weight)


# speedup vs baseline: 1.0309x; 1.0309x over previous
import jax
import jax.numpy as jnp
from jax.experimental import pallas as pl
from jax.experimental.pallas import tpu as pltpu

_BM = 512


def _gate_kernel(x_ref, w_ref, o_ref):
    o_ref[...] = jax.lax.dot_general(
        x_ref[...], w_ref[...], (((1,), (1,)), ((), ())),
        precision=jax.lax.Precision.DEFAULT,
        preferred_element_type=jnp.float32)


def kernel(hidden_states, weight):
    m, k = hidden_states.shape
    e = weight.shape[0]
    return pl.pallas_call(
        _gate_kernel,
        grid=(m // _BM,),
        in_specs=[
            pl.BlockSpec((_BM, k), lambda i: (i, 0)),
            pl.BlockSpec((e, k), lambda i: (0, 0)),
        ],
        out_specs=pl.BlockSpec((_BM, e), lambda i: (i, 0)),
        out_shape=jax.ShapeDtypeStruct((m, e), jnp.float32),
        compiler_params=pltpu.CompilerParams(
            dimension_semantics=(pltpu.PARALLEL,),
            disable_bounds_checks=True,
            skip_device_barrier=True),
    )(hidden_states, weight)
